# Initial kernel scaffold; baseline (speedup 1.0000x reference)
import jax
import jax.numpy as jnp
from jax.experimental import pallas as pl


def kernel(inputs, w_marg, w_cond):
    raise NotImplementedError("write your pallas kernel here")


# speedup vs baseline: 2.5998x; 2.5998x over previous
import functools

import jax
import jax.numpy as jnp
from jax import lax
from jax.experimental import pallas as pl
from jax.experimental.pallas import tpu as pltpu
from jax.experimental.pallas import tpu_sc as plsc

_NC = 2
_NS = 16
_NW = _NC * _NS
_LANES = 16

_CHUNK = 128


def _combo_body(wm_col_ref, wm_row_ref, wc_ref, out_ref):
    wc = wc_ref[...]
    m = jnp.max(wc, axis=1, keepdims=True)
    s = jnp.sum(jnp.exp(wc - m), axis=1, keepdims=True)
    lse_rows = jnp.log(s) + m
    wm_row = wm_row_ref[...]
    mm = jnp.max(wm_row)
    lse_marg = jnp.log(jnp.sum(jnp.exp(wm_row - mm))) + mm
    out_ref[...] = wm_col_ref[...] - lse_marg - lse_rows


def _combo_call(w_marg, w_cond):
    n = w_cond.shape[0]
    rows_blk = 512
    grid = (n // rows_blk,)
    return pl.pallas_call(
        _combo_body,
        grid=grid,
        in_specs=[
            pl.BlockSpec((rows_blk, 1), lambda i: (i, 0)),
            pl.BlockSpec((1, n), lambda i: (0, 0)),
            pl.BlockSpec((rows_blk, n), lambda i: (i, 0)),
        ],
        out_specs=pl.BlockSpec((rows_blk, 1), lambda i: (i, 0)),
        out_shape=jax.ShapeDtypeStruct((n, 1), jnp.float32),
    )(w_marg.reshape(n, 1), w_marg.reshape(1, n), w_cond)


def _sc_gather_body(n, rows_per_w, a_hbm, b_hbm, combo_hbm, wcflat_hbm,
                    out_hbm, a_v, fi_v, cv_v, wv_v, sem_c, sem_w):
    wid = lax.axis_index("s") * _NC + lax.axis_index("c")
    r0 = wid * rows_per_w
    pltpu.sync_copy(a_hbm.at[pl.ds(r0, rows_per_w)], a_v)
    pltpu.sync_copy(b_hbm.at[pl.ds(r0, rows_per_w)], fi_v)
    for r in range(rows_per_w):
        for j in range(_CHUNK // _LANES):
            sl = (r, pl.ds(j * _LANES, _LANES))
            fi_v[sl] = a_v[sl] * n + fi_v[sl]
    copies = []
    for r in range(rows_per_w):
        copies.append(pltpu.async_copy(combo_hbm.at[a_v.at[r]], cv_v.at[r], sem_c))
        copies.append(pltpu.async_copy(wcflat_hbm.at[fi_v.at[r]], wv_v.at[r], sem_w))
    for cp in copies:
        cp.wait()
    for r in range(rows_per_w):
        for j in range(_CHUNK // _LANES):
            sl = (r, pl.ds(j * _LANES, _LANES))
            wv_v[sl] = wv_v[sl] + cv_v[sl]
    pltpu.sync_copy(wv_v, out_hbm.at[pl.ds(r0, rows_per_w)])


def _sc_gather_call(a2, b2, combo, wcflat, n):
    num_rows = a2.shape[0]
    rows_per_w = num_rows // _NW
    mesh = plsc.VectorSubcoreMesh(
        core_axis_name="c", subcore_axis_name="s",
        num_cores=_NC, num_subcores=_NS)
    body = functools.partial(_sc_gather_body, n, rows_per_w)
    f = pl.kernel(
        body,
        out_type=jax.ShapeDtypeStruct((num_rows, _CHUNK), jnp.float32),
        mesh=mesh,
        scratch_types=[
            pltpu.VMEM((rows_per_w, _CHUNK), jnp.int32),
            pltpu.VMEM((rows_per_w, _CHUNK), jnp.int32),
            pltpu.VMEM((rows_per_w, _CHUNK), jnp.float32),
            pltpu.VMEM((rows_per_w, _CHUNK), jnp.float32),
            pltpu.SemaphoreType.DMA,
            pltpu.SemaphoreType.DMA,
        ],
    )
    return f(a2, b2, combo, wcflat)


def kernel(inputs, w_marg, w_cond):
    n = w_cond.shape[0]
    batch = inputs.shape[0]
    combo = _combo_call(w_marg, w_cond).reshape(n)
    a2 = inputs[:, 0].reshape(batch // _CHUNK, _CHUNK)
    b2 = inputs[:, 1].reshape(batch // _CHUNK, _CHUNK)
    wcflat = w_cond.reshape(n * n)
    out2 = _sc_gather_call(a2, b2, combo, wcflat, n)
    return out2.reshape(batch)


# speedup vs baseline: 6.2169x; 2.3913x over previous
import functools

import jax
import jax.numpy as jnp
from jax import lax
from jax.experimental import pallas as pl
from jax.experimental.pallas import tpu as pltpu
from jax.experimental.pallas import tpu_sc as plsc

_NC = 2
_NS = 16
_NW = _NC * _NS
_LANES = 16

_CHUNK = 128


def _combo_body(wm_col_ref, wm_row_ref, wc_ref, out_ref):
    wc = wc_ref[...]
    m = jnp.max(wc, axis=1, keepdims=True)
    s = jnp.sum(jnp.exp(wc - m), axis=1, keepdims=True)
    lse_rows = jnp.log(s) + m
    wm_row = wm_row_ref[...]
    mm = jnp.max(wm_row)
    lse_marg = jnp.log(jnp.sum(jnp.exp(wm_row - mm))) + mm
    out_ref[...] = wm_col_ref[...] - lse_marg - lse_rows


def _combo_call(w_marg, w_cond):
    n = w_cond.shape[0]
    rows_blk = 256
    grid = (n // rows_blk,)
    return pl.pallas_call(
        _combo_body,
        grid=grid,
        in_specs=[
            pl.BlockSpec((rows_blk, 1), lambda i: (i, 0)),
            pl.BlockSpec((1, n), lambda i: (0, 0)),
            pl.BlockSpec((rows_blk, n), lambda i: (i, 0)),
        ],
        out_specs=pl.BlockSpec((rows_blk, 1), lambda i: (i, 0)),
        out_shape=jax.ShapeDtypeStruct((n, 1), jnp.float32),
    )(w_marg.reshape(n, 1), w_marg.reshape(1, n), w_cond)


def _sc_gather_body(n, rows_per_w, a_hbm, b_hbm, combo_hbm, wcflat_hbm,
                    out_hbm, a_v, fi_v, cv_v, wv_v, sem_c, sem_w):
    wid = lax.axis_index("s") * _NC + lax.axis_index("c")
    r0 = wid * rows_per_w
    pltpu.sync_copy(a_hbm.at[pl.ds(r0, rows_per_w)], a_v)
    pltpu.sync_copy(b_hbm.at[pl.ds(r0, rows_per_w)], fi_v)
    ntile = n // 128
    for r in range(rows_per_w):
        for j in range(_CHUNK // _LANES):
            sl = (r, pl.ds(j * _LANES, _LANES))
            av = a_v[sl]
            bv = fi_v[sl]
            fi_v[sl] = (((av >> 3) * ntile + (bv >> 7)) * 1024
                        + ((av & 7) << 7) + (bv & 127))
    copies = []
    for r in range(rows_per_w):
        copies.append(pltpu.async_copy(combo_hbm.at[a_v.at[r]], cv_v.at[r], sem_c))
        copies.append(pltpu.async_copy(wcflat_hbm.at[fi_v.at[r]], wv_v.at[r], sem_w))
    for cp in copies:
        cp.wait()
    for r in range(rows_per_w):
        for j in range(_CHUNK // _LANES):
            sl = (r, pl.ds(j * _LANES, _LANES))
            wv_v[sl] = wv_v[sl] + cv_v[sl]
    pltpu.sync_copy(wv_v, out_hbm.at[pl.ds(r0, rows_per_w)])


def _sc_gather_call(a2, b2, combo, wcflat, n):
    num_rows = a2.shape[0]
    rows_per_w = num_rows // _NW
    mesh = plsc.VectorSubcoreMesh(
        core_axis_name="c", subcore_axis_name="s",
        num_cores=_NC, num_subcores=_NS)
    body = functools.partial(_sc_gather_body, n, rows_per_w)
    f = pl.kernel(
        body,
        out_type=jax.ShapeDtypeStruct((num_rows, _CHUNK), jnp.float32),
        mesh=mesh,
        scratch_types=[
            pltpu.VMEM((rows_per_w, _CHUNK), jnp.int32),
            pltpu.VMEM((rows_per_w, _CHUNK), jnp.int32),
            pltpu.VMEM((rows_per_w, _CHUNK), jnp.float32),
            pltpu.VMEM((rows_per_w, _CHUNK), jnp.float32),
            pltpu.SemaphoreType.DMA,
            pltpu.SemaphoreType.DMA,
        ],
    )
    return f(a2, b2, combo, wcflat)


def kernel(inputs, w_marg, w_cond):
    n = w_cond.shape[0]
    batch = inputs.shape[0]
    combo = _combo_call(w_marg, w_cond).reshape(n)
    a2 = inputs[:, 0].reshape(batch // _CHUNK, _CHUNK)
    b2 = inputs[:, 1].reshape(batch // _CHUNK, _CHUNK)
    wcflat = (w_cond.reshape(n // 8, 8, n // 128, 128)
              .transpose(0, 2, 1, 3).reshape(n * n))
    out2 = _sc_gather_call(a2, b2, combo, wcflat, n)
    return out2.reshape(batch)
